# baseline (speedup 1.0000x reference)
import functools

import jax
import jax.numpy as jnp
from jax.experimental import pallas as pl
from jax.experimental.pallas import tpu as pltpu


def _dots(a_ref, xc, wp_ref):
    p = jax.lax.dot_general(
        a_ref[0], xc, (((1,), (0,)), ((), ())),
        preferred_element_type=jnp.float32,
        precision=jax.lax.Precision.DEFAULT)
    return jax.lax.dot_general(
        p, wp_ref[...], (((1,), (0,)), ((), ())),
        preferred_element_type=jnp.float32)


def _fused_block(a1a_ref, a1b_ref, a2a_ref, a2b_ref, x_ref, w1p_ref,
                 w2p_ref, b_ref, out_ref):
    bh = a1a_ref.shape[1]
    xc = x_ref[...]
    out_ref[:bh] = (_dots(a1a_ref, xc, w1p_ref) +
                    _dots(a2a_ref, xc, w2p_ref) + b_ref[...])
    out_ref[bh:] = (_dots(a1b_ref, xc, w1p_ref) +
                    _dots(a2b_ref, xc, w2p_ref) + b_ref[...])


@functools.partial(jax.jit, static_argnames=())
def kernel(x, adj1, adj2, W, b):
    n, two_dim = x.shape
    dim = two_dim // 2
    out_f = W.shape[0]

    wt = W.T
    zeros = jnp.zeros((dim, out_f), wt.dtype)
    w1p = jnp.concatenate([wt[:dim, :], zeros], axis=0)
    w2p = jnp.concatenate([zeros, wt[dim:, :]], axis=0)
    b2 = b.reshape(1, out_f)

    bh = 100 if n % 200 == 0 else (n // 2 if (n // 2) % 8 == 0 else n)
    bm = 2 * bh
    grid = (n // bm,)
    a3 = (n // bh, bh, n)
    adj1r = adj1.reshape(a3)
    adj2r = adj2.reshape(a3)

    return pl.pallas_call(
        _fused_block,
        grid=grid,
        in_specs=[
            pl.BlockSpec((1, bh, n), lambda i: (2 * i, 0, 0)),
            pl.BlockSpec((1, bh, n), lambda i: (2 * i + 1, 0, 0)),
            pl.BlockSpec((1, bh, n), lambda i: (2 * i, 0, 0)),
            pl.BlockSpec((1, bh, n), lambda i: (2 * i + 1, 0, 0)),
            pl.BlockSpec((n, two_dim), lambda i: (0, 0)),
            pl.BlockSpec((two_dim, out_f), lambda i: (0, 0)),
            pl.BlockSpec((two_dim, out_f), lambda i: (0, 0)),
            pl.BlockSpec((1, out_f), lambda i: (0, 0)),
        ],
        out_specs=pl.BlockSpec((bm, out_f), lambda i: (i, 0)),
        out_shape=jax.ShapeDtypeStruct((n, out_f), jnp.float32),
        compiler_params=pltpu.CompilerParams(
            dimension_semantics=("arbitrary",),
            vmem_limit_bytes=63 * 1024 * 1024,
        ),
    )(adj1r, adj1r, adj2r, adj2r, x, w1p, w2p, b2)


# speedup vs baseline: 4.0152x; 4.0152x over previous
import functools

import jax
import jax.numpy as jnp
from jax.experimental import pallas as pl
from jax.experimental.pallas import tpu as pltpu


def _fused_block(adj1_ref, adj2_ref, x_ref, w1p_ref, w2p_ref, b_ref,
                 out_ref):
    xc = x_ref[...]
    p1 = jax.lax.dot_general(
        adj1_ref[...], xc,
        (((1,), (0,)), ((), ())), preferred_element_type=jnp.float32,
        precision=jax.lax.Precision.DEFAULT)
    p2 = jax.lax.dot_general(
        adj2_ref[...], xc,
        (((1,), (0,)), ((), ())), preferred_element_type=jnp.float32,
        precision=jax.lax.Precision.DEFAULT)
    o = jax.lax.dot_general(
        p1, w1p_ref[...], (((1,), (0,)), ((), ())),
        preferred_element_type=jnp.float32)
    o += jax.lax.dot_general(
        p2, w2p_ref[...], (((1,), (0,)), ((), ())),
        preferred_element_type=jnp.float32)
    out_ref[...] = o + b_ref[...]


@functools.partial(jax.jit, static_argnames=())
def kernel(x, adj1, adj2, W, b):
    n, two_dim = x.shape
    dim = two_dim // 2
    out_f = W.shape[0]

    wt = W.T
    zeros = jnp.zeros((dim, out_f), wt.dtype)
    w1p = jnp.concatenate([wt[:dim, :], zeros], axis=0)
    w2p = jnp.concatenate([zeros, wt[dim:, :]], axis=0)
    b2 = b.reshape(1, out_f)

    bm = 200 if n % 200 == 0 else (8 if n % 8 == 0 else n)
    grid = (n // bm,)

    return pl.pallas_call(
        _fused_block,
        grid=grid,
        in_specs=[
            pl.BlockSpec((bm, n), lambda i: (i, 0)),
            pl.BlockSpec((bm, n), lambda i: (i, 0)),
            pl.BlockSpec((n, two_dim), lambda i: (0, 0)),
            pl.BlockSpec((two_dim, out_f), lambda i: (0, 0)),
            pl.BlockSpec((two_dim, out_f), lambda i: (0, 0)),
            pl.BlockSpec((1, out_f), lambda i: (0, 0)),
        ],
        out_specs=pl.BlockSpec((bm, out_f), lambda i: (i, 0)),
        out_shape=jax.ShapeDtypeStruct((n, out_f), jnp.float32),
        compiler_params=pltpu.CompilerParams(
            dimension_semantics=("arbitrary",),
            vmem_limit_bytes=63 * 1024 * 1024,
        ),
    )(adj1, adj2, x, w1p, w2p, b2)
